# Initial kernel scaffold; baseline (speedup 1.0000x reference)
import functools

import jax
import jax.numpy as jnp
from jax import lax
from jax.experimental import pallas as pl
from jax.experimental.pallas import tpu as pltpu
from jax.experimental.pallas import tpu_sc as plsc

N = 10000
E = 320000
D = 128
G = 64
NP = 10112
EB = 2560
EPAD = EB * 128 - E
NC = 2
NS = 16
WPW = EB // (NC * NS)
RPS = NP // NS

_mesh = plsc.VectorSubcoreMesh(core_axis_name="c", subcore_axis_name="s")
_HIGH = jax.lax.Precision.HIGHEST


@functools.partial(
    pl.kernel,
    out_type=jax.ShapeDtypeStruct((NC, NP, 16), jnp.float32),
    mesh=_mesh,
    scratch_types=[
        pltpu.VMEM_SHARED((NP, 16), jnp.float32),
        pltpu.VMEM((WPW, 128), jnp.int32),
        pltpu.VMEM((128, 16), jnp.float32),
    ],
)
def _deg_kernel(dst_hbm, ones_hbm, zeros_hbm, out_hbm, acc, didx, ones_v):
    c = lax.axis_index("c")
    s = lax.axis_index("s")
    wid = s * NC + c
    pltpu.sync_copy(zeros_hbm.at[pl.ds(s * RPS, RPS)], acc.at[pl.ds(s * RPS, RPS)])
    pltpu.sync_copy(ones_hbm, ones_v)
    pltpu.sync_copy(dst_hbm.at[pl.ds(wid * WPW, WPW)], didx)
    plsc.subcore_barrier()

    @pl.loop(0, WPW)
    def _(j):
        pltpu.sync_copy(ones_v, acc.at[didx.at[j]], add=True)

    plsc.subcore_barrier()
    pltpu.sync_copy(acc.at[pl.ds(s * RPS, RPS)], out_hbm.at[c, pl.ds(s * RPS, RPS)])


@functools.partial(
    pl.kernel,
    out_type=jax.ShapeDtypeStruct((NC, NP, D), jnp.float32),
    mesh=_mesh,
    scratch_types=[
        pltpu.VMEM_SHARED((NP, D), jnp.float32),
        pltpu.VMEM((WPW, 128), jnp.int32),
        pltpu.VMEM((WPW, 128), jnp.int32),
        pltpu.VMEM((128, D), jnp.float32),
    ],
)
def _spmm_kernel(g_hbm, src_hbm, dst_hbm, zeros_hbm, out_hbm, acc, sidx, didx, rows):
    c = lax.axis_index("c")
    s = lax.axis_index("s")
    wid = s * NC + c
    pltpu.sync_copy(zeros_hbm.at[pl.ds(s * RPS, RPS)], acc.at[pl.ds(s * RPS, RPS)])
    pltpu.sync_copy(src_hbm.at[pl.ds(wid * WPW, WPW)], sidx)
    pltpu.sync_copy(dst_hbm.at[pl.ds(wid * WPW, WPW)], didx)
    plsc.subcore_barrier()

    @pl.loop(0, WPW)
    def _(j):
        pltpu.sync_copy(g_hbm.at[sidx.at[j]], rows)
        pltpu.sync_copy(rows, acc.at[didx.at[j]], add=True)

    plsc.subcore_barrier()
    pltpu.sync_copy(acc.at[pl.ds(s * RPS, RPS)], out_hbm.at[c, pl.ds(s * RPS, RPS)])


def _tc1_body(x_ref, h0_ref, h1_ref, w_ref, g_ref, dinv_ref):
    deg = 1.0 + h0_ref[:, 0:1] + h1_ref[:, 0:1]
    dinv = lax.rsqrt(jnp.maximum(deg, 1.0))
    g_ref[...] = jnp.dot(x_ref[...], w_ref[...],
                         preferred_element_type=jnp.float32, precision=_HIGH) * dinv
    dinv_ref[...] = dinv


_tc1 = pl.pallas_call(
    _tc1_body,
    grid=(NP // 128,),
    in_specs=[
        pl.BlockSpec((128, D), lambda i: (i, 0)),
        pl.BlockSpec((128, 16), lambda i: (i, 0)),
        pl.BlockSpec((128, 16), lambda i: (i, 0)),
        pl.BlockSpec((D, D), lambda i: (0, 0)),
    ],
    out_specs=[
        pl.BlockSpec((128, D), lambda i: (i, 0)),
        pl.BlockSpec((128, 1), lambda i: (i, 0)),
    ],
    out_shape=[
        jax.ShapeDtypeStruct((NP, D), jnp.float32),
        jax.ShapeDtypeStruct((NP, 1), jnp.float32),
    ],
)


def _tc2_body(sa_ref, sb_ref, g1_ref, dinv_ref, b_ref, w_ref, g2_ref):
    dinv = dinv_ref[...]
    h1 = jnp.maximum(dinv * (sa_ref[...] + sb_ref[...] + g1_ref[...]) + b_ref[...], 0.0)
    g2_ref[...] = jnp.dot(h1, w_ref[...],
                          preferred_element_type=jnp.float32, precision=_HIGH) * dinv


_tc2 = pl.pallas_call(
    _tc2_body,
    grid=(NP // 128,),
    in_specs=[
        pl.BlockSpec((128, D), lambda i: (i, 0)),
        pl.BlockSpec((128, D), lambda i: (i, 0)),
        pl.BlockSpec((128, D), lambda i: (i, 0)),
        pl.BlockSpec((128, 1), lambda i: (i, 0)),
        pl.BlockSpec((1, D), lambda i: (0, 0)),
        pl.BlockSpec((D, D), lambda i: (0, 0)),
    ],
    out_specs=pl.BlockSpec((128, D), lambda i: (i, 0)),
    out_shape=jax.ShapeDtypeStruct((NP, D), jnp.float32),
)


def _tc3_body(sa_ref, sb_ref, g2_ref, dinv_ref, b_ref, batch_ref, wl_ref, bl_ref, out_ref):
    dinv = dinv_ref[...]
    h2 = jnp.maximum(dinv * (sa_ref[...] + sb_ref[...] + g2_ref[...]) + b_ref[...], 0.0)
    oh = (lax.broadcasted_iota(jnp.int32, (G, NP), 0)
          == batch_ref[...]).astype(jnp.float32)
    sums = lax.dot_general(oh, h2, (((1,), (0,)), ((), ())),
                           preferred_element_type=jnp.float32, precision=_HIGH)
    counts = jnp.sum(oh, axis=1).reshape(G, 1)
    pooled = sums / jnp.maximum(counts, 1.0)
    out_ref[...] = jnp.dot(pooled, wl_ref[...],
                           preferred_element_type=jnp.float32, precision=_HIGH) + bl_ref[...]


_tc3 = pl.pallas_call(
    _tc3_body,
    in_specs=[
        pl.BlockSpec((NP, D), lambda: (0, 0)),
        pl.BlockSpec((NP, D), lambda: (0, 0)),
        pl.BlockSpec((NP, D), lambda: (0, 0)),
        pl.BlockSpec((NP, 1), lambda: (0, 0)),
        pl.BlockSpec((1, D), lambda: (0, 0)),
        pl.BlockSpec((1, NP), lambda: (0, 0)),
        pl.BlockSpec((D, 1), lambda: (0, 0)),
        pl.BlockSpec((1, 1), lambda: (0, 0)),
    ],
    out_specs=pl.BlockSpec((G, 1), lambda: (0, 0)),
    out_shape=jax.ShapeDtypeStruct((G, 1), jnp.float32),
)


def kernel(x, edge_index, batch, W1, b1, W2, b2, Wl, bl):
    x = x.astype(jnp.float32)
    src = edge_index[0]
    dst = edge_index[1]
    pad_i = jnp.arange(EPAD, dtype=src.dtype)
    src2d = jnp.concatenate([src, pad_i % 128]).reshape(EB, 128)
    dst2d = jnp.concatenate([dst, N + pad_i % (NP - N)]).reshape(EB, 128)
    x_pad = jnp.pad(x, ((0, NP - N), (0, 0)))
    batch_pad = jnp.pad(batch.astype(jnp.int32), (0, NP - N),
                        constant_values=G).reshape(1, NP)
    zeros16 = jnp.zeros((NP, 16), jnp.float32)
    zeros128 = jnp.zeros((NP, D), jnp.float32)
    ones16 = jnp.ones((128, 16), jnp.float32)

    hist = _deg_kernel(dst2d, ones16, zeros16)
    g1, dinv = _tc1(x_pad, hist[0], hist[1], W1)
    s1 = _spmm_kernel(g1, src2d, dst2d, zeros128)
    g2 = _tc2(s1[0], s1[1], g1, dinv, b1.reshape(1, D), W2)
    s2 = _spmm_kernel(g2, src2d, dst2d, zeros128)
    out = _tc3(s2[0], s2[1], g2, dinv, b2.reshape(1, D), batch_pad,
               Wl, bl.reshape(1, 1))
    return out.reshape(-1)


# speedup vs baseline: 16.8051x; 16.8051x over previous
import functools

import jax
import jax.numpy as jnp
from jax import lax
from jax.experimental import pallas as pl
from jax.experimental.pallas import tpu as pltpu
from jax.experimental.pallas import tpu_sc as plsc

N = 10000
E = 320000
D = 128
G = 64
NP = 10112
EB = 2560
EPAD = EB * 128 - E
NC = 2
NS = 16
WPW = EB // (NC * NS)
RPS = NP // NS

_mesh = plsc.VectorSubcoreMesh(core_axis_name="c", subcore_axis_name="s")
_HIGH = jax.lax.Precision.HIGHEST


@functools.partial(
    pl.kernel,
    out_type=jax.ShapeDtypeStruct((NC, NP, D), jnp.float32),
    mesh=_mesh,
    scratch_types=[
        pltpu.VMEM_SHARED((NP, D), jnp.float32),
        pltpu.VMEM((WPW, 128), jnp.int32),
        pltpu.VMEM((128, D), jnp.float32),
    ],
)
def _deg_kernel(dst_hbm, ones_hbm, zeros_hbm, out_hbm, acc, didx, ones_v):
    c = lax.axis_index("c")
    s = lax.axis_index("s")
    wid = s * NC + c
    pltpu.sync_copy(zeros_hbm.at[pl.ds(s * RPS, RPS)], acc.at[pl.ds(s * RPS, RPS)])
    pltpu.sync_copy(ones_hbm, ones_v)
    pltpu.sync_copy(dst_hbm.at[pl.ds(wid * WPW, WPW)], didx)
    plsc.subcore_barrier()

    @pl.loop(0, WPW)
    def _(j):
        pltpu.sync_copy(ones_v, acc.at[didx.at[j]], add=True)

    plsc.subcore_barrier()
    pltpu.sync_copy(acc.at[pl.ds(s * RPS, RPS)], out_hbm.at[c, pl.ds(s * RPS, RPS)])


@functools.partial(
    pl.kernel,
    out_type=jax.ShapeDtypeStruct((NC, NP, D), jnp.float32),
    mesh=_mesh,
    scratch_types=[
        pltpu.VMEM_SHARED((NP, D), jnp.float32),
        pltpu.VMEM((WPW, 128), jnp.int32),
        pltpu.VMEM((WPW, 128), jnp.int32),
        pltpu.VMEM((128, D), jnp.float32),
    ],
)
def _spmm_kernel(g_hbm, src_hbm, dst_hbm, zeros_hbm, out_hbm, acc, sidx, didx, rows):
    c = lax.axis_index("c")
    s = lax.axis_index("s")
    wid = s * NC + c
    pltpu.sync_copy(zeros_hbm.at[pl.ds(s * RPS, RPS)], acc.at[pl.ds(s * RPS, RPS)])
    pltpu.sync_copy(src_hbm.at[pl.ds(wid * WPW, WPW)], sidx)
    pltpu.sync_copy(dst_hbm.at[pl.ds(wid * WPW, WPW)], didx)
    plsc.subcore_barrier()

    @pl.loop(0, WPW)
    def _(j):
        pltpu.sync_copy(g_hbm.at[sidx.at[j]], rows)
        pltpu.sync_copy(rows, acc.at[didx.at[j]], add=True)

    plsc.subcore_barrier()
    pltpu.sync_copy(acc.at[pl.ds(s * RPS, RPS)], out_hbm.at[c, pl.ds(s * RPS, RPS)])


def _tc1_body(x_ref, h0_ref, h1_ref, w_ref, g_ref, dinv_ref):
    deg = 1.0 + h0_ref[:, 0:1] + h1_ref[:, 0:1]
    dinv = lax.rsqrt(jnp.maximum(deg, 1.0))
    g_ref[...] = jnp.dot(x_ref[...], w_ref[...],
                         preferred_element_type=jnp.float32, precision=_HIGH) * dinv
    dinv_ref[...] = dinv


_tc1 = pl.pallas_call(
    _tc1_body,
    grid=(NP // 128,),
    in_specs=[
        pl.BlockSpec((128, D), lambda i: (i, 0)),
        pl.BlockSpec((128, D), lambda i: (i, 0)),
        pl.BlockSpec((128, D), lambda i: (i, 0)),
        pl.BlockSpec((D, D), lambda i: (0, 0)),
    ],
    out_specs=[
        pl.BlockSpec((128, D), lambda i: (i, 0)),
        pl.BlockSpec((128, 1), lambda i: (i, 0)),
    ],
    out_shape=[
        jax.ShapeDtypeStruct((NP, D), jnp.float32),
        jax.ShapeDtypeStruct((NP, 1), jnp.float32),
    ],
)


def _tc2_body(sa_ref, sb_ref, g1_ref, dinv_ref, b_ref, w_ref, g2_ref):
    dinv = dinv_ref[...]
    h1 = jnp.maximum(dinv * (sa_ref[...] + sb_ref[...] + g1_ref[...]) + b_ref[...], 0.0)
    g2_ref[...] = jnp.dot(h1, w_ref[...],
                          preferred_element_type=jnp.float32, precision=_HIGH) * dinv


_tc2 = pl.pallas_call(
    _tc2_body,
    grid=(NP // 128,),
    in_specs=[
        pl.BlockSpec((128, D), lambda i: (i, 0)),
        pl.BlockSpec((128, D), lambda i: (i, 0)),
        pl.BlockSpec((128, D), lambda i: (i, 0)),
        pl.BlockSpec((128, 1), lambda i: (i, 0)),
        pl.BlockSpec((1, D), lambda i: (0, 0)),
        pl.BlockSpec((D, D), lambda i: (0, 0)),
    ],
    out_specs=pl.BlockSpec((128, D), lambda i: (i, 0)),
    out_shape=jax.ShapeDtypeStruct((NP, D), jnp.float32),
)


def _tc3_body(sa_ref, sb_ref, g2_ref, dinv_ref, b_ref, batch_ref, wl_ref, bl_ref, out_ref):
    dinv = dinv_ref[...]
    h2 = jnp.maximum(dinv * (sa_ref[...] + sb_ref[...] + g2_ref[...]) + b_ref[...], 0.0)
    oh = (lax.broadcasted_iota(jnp.int32, (G, NP), 0)
          == batch_ref[...]).astype(jnp.float32)
    sums = lax.dot_general(oh, h2, (((1,), (0,)), ((), ())),
                           preferred_element_type=jnp.float32, precision=_HIGH)
    counts = jnp.sum(oh, axis=1).reshape(G, 1)
    pooled = sums / jnp.maximum(counts, 1.0)
    out_ref[...] = jnp.dot(pooled, wl_ref[...],
                           preferred_element_type=jnp.float32, precision=_HIGH) + bl_ref[...]


_tc3 = pl.pallas_call(
    _tc3_body,
    in_specs=[
        pl.BlockSpec((NP, D), lambda: (0, 0)),
        pl.BlockSpec((NP, D), lambda: (0, 0)),
        pl.BlockSpec((NP, D), lambda: (0, 0)),
        pl.BlockSpec((NP, 1), lambda: (0, 0)),
        pl.BlockSpec((1, D), lambda: (0, 0)),
        pl.BlockSpec((1, NP), lambda: (0, 0)),
        pl.BlockSpec((D, 1), lambda: (0, 0)),
        pl.BlockSpec((1, 1), lambda: (0, 0)),
    ],
    out_specs=pl.BlockSpec((G, 1), lambda: (0, 0)),
    out_shape=jax.ShapeDtypeStruct((G, 1), jnp.float32),
)


def kernel(x, edge_index, batch, W1, b1, W2, b2, Wl, bl):
    x = x.astype(jnp.float32)
    src = edge_index[0]
    dst = edge_index[1]
    pad_i = jnp.arange(EPAD, dtype=src.dtype)
    src2d = jnp.concatenate([src, pad_i % 128]).reshape(EB, 128)
    dst2d = jnp.concatenate([dst, N + pad_i % (NP - N)]).reshape(EB, 128)
    x_pad = jnp.pad(x, ((0, NP - N), (0, 0)))
    batch_pad = jnp.pad(batch.astype(jnp.int32), (0, NP - N),
                        constant_values=G).reshape(1, NP)
    zeros128 = jnp.zeros((NP, D), jnp.float32)
    ones128 = jnp.ones((128, D), jnp.float32)

    hist = _deg_kernel(dst2d, ones128, zeros128)
    g1, dinv = _tc1(x_pad, hist[0], hist[1], W1)
    s1 = _spmm_kernel(g1, src2d, dst2d, zeros128)
    g2 = _tc2(s1[0], s1[1], g1, dinv, b1.reshape(1, D), W2)
    s2 = _spmm_kernel(g2, src2d, dst2d, zeros128)
    out = _tc3(s2[0], s2[1], g2, dinv, b2.reshape(1, D), batch_pad,
               Wl, bl.reshape(1, 1))
    return out.reshape(-1)


# speedup vs baseline: 19.7702x; 1.1764x over previous
import functools

import jax
import jax.numpy as jnp
from jax import lax
from jax.experimental import pallas as pl
from jax.experimental.pallas import tpu as pltpu
from jax.experimental.pallas import tpu_sc as plsc

N = 10000
E = 320000
D = 128
G = 64
NP = 10112
EB = 2560
EPAD = EB * 128 - E
NC = 2
NS = 16
WPW = EB // (NC * NS)
RPS = NP // NS

_mesh = plsc.VectorSubcoreMesh(core_axis_name="c", subcore_axis_name="s")
_HIGH = jax.lax.Precision.HIGHEST


@functools.partial(
    pl.kernel,
    out_type=jax.ShapeDtypeStruct((NC, NP, D), jnp.float32),
    mesh=_mesh,
    scratch_types=[
        pltpu.VMEM_SHARED((NP, D), jnp.float32),
        pltpu.VMEM((WPW, 128), jnp.int32),
        pltpu.VMEM((128, D), jnp.float32),
    ],
)
def _deg_kernel(dst_hbm, ones_hbm, zeros_hbm, out_hbm, acc, didx, ones_v):
    c = lax.axis_index("c")
    s = lax.axis_index("s")
    wid = s * NC + c
    pltpu.sync_copy(zeros_hbm.at[pl.ds(s * RPS, RPS)], acc.at[pl.ds(s * RPS, RPS)])
    pltpu.sync_copy(ones_hbm, ones_v)
    pltpu.sync_copy(dst_hbm.at[pl.ds(wid * WPW, WPW)], didx)
    plsc.subcore_barrier()

    @pl.loop(0, WPW)
    def _(j):
        pltpu.sync_copy(ones_v, acc.at[didx.at[j]], add=True)

    plsc.subcore_barrier()
    pltpu.sync_copy(acc.at[pl.ds(s * RPS, RPS)], out_hbm.at[c, pl.ds(s * RPS, RPS)])


@functools.partial(
    pl.kernel,
    out_type=jax.ShapeDtypeStruct((NC, NP, D), jnp.float32),
    mesh=_mesh,
    scratch_types=[
        pltpu.VMEM_SHARED((NP, D), jnp.float32),
        pltpu.VMEM((40, 128), jnp.int32),
        pltpu.VMEM((40, 128), jnp.int32),
        pltpu.VMEM((128, D), jnp.float32),
        pltpu.VMEM((128, D), jnp.float32),
        pltpu.SemaphoreType.DMA,
        pltpu.SemaphoreType.DMA,
    ],
)
def _spmm_kernel(g_hbm, src_hbm, dst_hbm, zeros_hbm, out_hbm, acc, sidx, didx,
                 rows0, rows1, sem0, sem1):
    c = lax.axis_index("c")
    s = lax.axis_index("s")
    wid = s * NC + c
    pltpu.sync_copy(zeros_hbm.at[pl.ds(s * RPS, RPS)], acc.at[pl.ds(s * RPS, RPS)])
    plsc.subcore_barrier()

    @pl.loop(0, WPW // 40)
    def _(t):
        base = wid * WPW + t * 40
        pltpu.sync_copy(src_hbm.at[pl.ds(base, 40)], sidx)
        pltpu.sync_copy(dst_hbm.at[pl.ds(base, 40)], didx)
        pltpu.async_copy(g_hbm.at[sidx.at[0]], rows0, sem0)

        @pl.loop(0, 40, step=2)
        def _(j):
            pltpu.make_async_copy(g_hbm.at[sidx.at[j]], rows0, sem0).wait()
            pltpu.async_copy(g_hbm.at[sidx.at[j + 1]], rows1, sem1)
            pltpu.sync_copy(rows0, acc.at[didx.at[j]], add=True)
            pltpu.make_async_copy(g_hbm.at[sidx.at[j + 1]], rows1, sem1).wait()

            @pl.when(j + 2 < 40)
            def _():
                pltpu.async_copy(g_hbm.at[sidx.at[j + 2]], rows0, sem0)

            pltpu.sync_copy(rows1, acc.at[didx.at[j + 1]], add=True)

    plsc.subcore_barrier()
    pltpu.sync_copy(acc.at[pl.ds(s * RPS, RPS)], out_hbm.at[c, pl.ds(s * RPS, RPS)])


def _tc1_body(x_ref, h0_ref, h1_ref, w_ref, g_ref, dinv_ref):
    deg = 1.0 + h0_ref[:, 0:1] + h1_ref[:, 0:1]
    dinv = lax.rsqrt(jnp.maximum(deg, 1.0))
    g_ref[...] = jnp.dot(x_ref[...], w_ref[...],
                         preferred_element_type=jnp.float32, precision=_HIGH) * dinv
    dinv_ref[...] = dinv


_tc1 = pl.pallas_call(
    _tc1_body,
    grid=(NP // 128,),
    in_specs=[
        pl.BlockSpec((128, D), lambda i: (i, 0)),
        pl.BlockSpec((128, D), lambda i: (i, 0)),
        pl.BlockSpec((128, D), lambda i: (i, 0)),
        pl.BlockSpec((D, D), lambda i: (0, 0)),
    ],
    out_specs=[
        pl.BlockSpec((128, D), lambda i: (i, 0)),
        pl.BlockSpec((128, 1), lambda i: (i, 0)),
    ],
    out_shape=[
        jax.ShapeDtypeStruct((NP, D), jnp.float32),
        jax.ShapeDtypeStruct((NP, 1), jnp.float32),
    ],
)


def _tc2_body(sa_ref, sb_ref, g1_ref, dinv_ref, b_ref, w_ref, g2_ref):
    dinv = dinv_ref[...]
    h1 = jnp.maximum(dinv * (sa_ref[...] + sb_ref[...] + g1_ref[...]) + b_ref[...], 0.0)
    g2_ref[...] = jnp.dot(h1, w_ref[...],
                          preferred_element_type=jnp.float32, precision=_HIGH) * dinv


_tc2 = pl.pallas_call(
    _tc2_body,
    grid=(NP // 128,),
    in_specs=[
        pl.BlockSpec((128, D), lambda i: (i, 0)),
        pl.BlockSpec((128, D), lambda i: (i, 0)),
        pl.BlockSpec((128, D), lambda i: (i, 0)),
        pl.BlockSpec((128, 1), lambda i: (i, 0)),
        pl.BlockSpec((1, D), lambda i: (0, 0)),
        pl.BlockSpec((D, D), lambda i: (0, 0)),
    ],
    out_specs=pl.BlockSpec((128, D), lambda i: (i, 0)),
    out_shape=jax.ShapeDtypeStruct((NP, D), jnp.float32),
)


def _tc3_body(sa_ref, sb_ref, g2_ref, dinv_ref, b_ref, batch_ref, wl_ref, bl_ref, out_ref):
    dinv = dinv_ref[...]
    h2 = jnp.maximum(dinv * (sa_ref[...] + sb_ref[...] + g2_ref[...]) + b_ref[...], 0.0)
    oh = (lax.broadcasted_iota(jnp.int32, (G, NP), 0)
          == batch_ref[...]).astype(jnp.float32)
    sums = lax.dot_general(oh, h2, (((1,), (0,)), ((), ())),
                           preferred_element_type=jnp.float32, precision=_HIGH)
    counts = jnp.sum(oh, axis=1).reshape(G, 1)
    pooled = sums / jnp.maximum(counts, 1.0)
    out_ref[...] = jnp.dot(pooled, wl_ref[...],
                           preferred_element_type=jnp.float32, precision=_HIGH) + bl_ref[...]


_tc3 = pl.pallas_call(
    _tc3_body,
    in_specs=[
        pl.BlockSpec((NP, D), lambda: (0, 0)),
        pl.BlockSpec((NP, D), lambda: (0, 0)),
        pl.BlockSpec((NP, D), lambda: (0, 0)),
        pl.BlockSpec((NP, 1), lambda: (0, 0)),
        pl.BlockSpec((1, D), lambda: (0, 0)),
        pl.BlockSpec((1, NP), lambda: (0, 0)),
        pl.BlockSpec((D, 1), lambda: (0, 0)),
        pl.BlockSpec((1, 1), lambda: (0, 0)),
    ],
    out_specs=pl.BlockSpec((G, 1), lambda: (0, 0)),
    out_shape=jax.ShapeDtypeStruct((G, 1), jnp.float32),
)


def kernel(x, edge_index, batch, W1, b1, W2, b2, Wl, bl):
    x = x.astype(jnp.float32)
    src = edge_index[0]
    dst = edge_index[1]
    pad_i = jnp.arange(EPAD, dtype=src.dtype)
    src2d = jnp.concatenate([src, pad_i % 128]).reshape(EB, 128)
    dst2d = jnp.concatenate([dst, N + pad_i % (NP - N)]).reshape(EB, 128)
    x_pad = jnp.pad(x, ((0, NP - N), (0, 0)))
    batch_pad = jnp.pad(batch.astype(jnp.int32), (0, NP - N),
                        constant_values=G).reshape(1, NP)
    zeros128 = jnp.zeros((NP, D), jnp.float32)
    ones128 = jnp.ones((128, D), jnp.float32)

    hist = _deg_kernel(dst2d, ones128, zeros128)
    g1, dinv = _tc1(x_pad, hist[0], hist[1], W1)
    s1 = _spmm_kernel(g1, src2d, dst2d, zeros128)
    g2 = _tc2(s1[0], s1[1], g1, dinv, b1.reshape(1, D), W2)
    s2 = _spmm_kernel(g2, src2d, dst2d, zeros128)
    out = _tc3(s2[0], s2[1], g2, dinv, b2.reshape(1, D), batch_pad,
               Wl, bl.reshape(1, 1))
    return out.reshape(-1)


# speedup vs baseline: 24.4016x; 1.2343x over previous
import functools

import jax
import jax.numpy as jnp
from jax import lax
from jax.experimental import pallas as pl
from jax.experimental.pallas import tpu as pltpu
from jax.experimental.pallas import tpu_sc as plsc

N = 10000
E = 320000
D = 128
G = 64
NP = 10112
EB = 2560
EPAD = EB * 128 - E
NC = 2
NS = 16
WPW = EB // (NC * NS)
RPS = NP // NS

_mesh = plsc.VectorSubcoreMesh(core_axis_name="c", subcore_axis_name="s")
_HIGH = jax.lax.Precision.HIGHEST


@functools.partial(
    pl.kernel,
    out_type=jax.ShapeDtypeStruct((NC, NP, D), jnp.float32),
    mesh=_mesh,
    scratch_types=[
        pltpu.VMEM_SHARED((NP, D), jnp.float32),
        pltpu.VMEM((WPW, 128), jnp.int32),
        pltpu.VMEM((128, D), jnp.float32),
    ],
)
def _deg_kernel(dst_hbm, ones_hbm, zeros_hbm, out_hbm, acc, didx, ones_v):
    c = lax.axis_index("c")
    s = lax.axis_index("s")
    wid = s * NC + c
    pltpu.sync_copy(zeros_hbm.at[pl.ds(s * RPS, RPS)], acc.at[pl.ds(s * RPS, RPS)])
    pltpu.sync_copy(ones_hbm, ones_v)
    pltpu.sync_copy(dst_hbm.at[pl.ds(wid * WPW, WPW)], didx)
    plsc.subcore_barrier()

    @pl.loop(0, WPW)
    def _(j):
        pltpu.sync_copy(ones_v, acc.at[didx.at[j]], add=True)

    plsc.subcore_barrier()
    pltpu.sync_copy(acc.at[pl.ds(s * RPS, RPS)], out_hbm.at[c, pl.ds(s * RPS, RPS)])


@functools.partial(
    pl.kernel,
    out_type=jax.ShapeDtypeStruct((NC, NP, D), jnp.float32),
    mesh=_mesh,
    scratch_types=[
        pltpu.VMEM_SHARED((NP, D), jnp.float32),
        pltpu.VMEM((40, 128), jnp.int32),
        pltpu.VMEM((40, 128), jnp.int32),
        pltpu.VMEM((128, D), jnp.float32),
        pltpu.VMEM((128, D), jnp.float32),
        pltpu.SemaphoreType.DMA,
        pltpu.SemaphoreType.DMA,
    ],
)
def _spmm_kernel(g_hbm, src_hbm, dst_hbm, zeros_hbm, out_hbm, acc, sidx, didx,
                 rows0, rows1, sem0, sem1):
    c = lax.axis_index("c")
    s = lax.axis_index("s")
    wid = s * NC + c
    pltpu.sync_copy(zeros_hbm.at[pl.ds(s * RPS, RPS)], acc.at[pl.ds(s * RPS, RPS)])
    plsc.subcore_barrier()

    @pl.loop(0, WPW // 40)
    def _(t):
        base = wid * WPW + t * 40
        pltpu.sync_copy(src_hbm.at[pl.ds(base, 40)], sidx)
        pltpu.sync_copy(dst_hbm.at[pl.ds(base, 40)], didx)
        pltpu.async_copy(g_hbm.at[sidx.at[0]], rows0, sem0)

        @pl.loop(0, 40, step=2)
        def _(j):
            pltpu.make_async_copy(g_hbm.at[sidx.at[j]], rows0, sem0).wait()
            pltpu.async_copy(g_hbm.at[sidx.at[j + 1]], rows1, sem1)
            pltpu.sync_copy(rows0, acc.at[didx.at[j]], add=True)
            pltpu.make_async_copy(g_hbm.at[sidx.at[j + 1]], rows1, sem1).wait()

            @pl.when(j + 2 < 40)
            def _():
                pltpu.async_copy(g_hbm.at[sidx.at[j + 2]], rows0, sem0)

            pltpu.sync_copy(rows1, acc.at[didx.at[j + 1]], add=True)

    plsc.subcore_barrier()
    pltpu.sync_copy(acc.at[pl.ds(s * RPS, RPS)], out_hbm.at[c, pl.ds(s * RPS, RPS)])


BR = 1264


def _tc1a_body(x_ref, w_ref, g_ref):
    g_ref[...] = jnp.dot(x_ref[...], w_ref[...],
                         preferred_element_type=jnp.float32, precision=_HIGH)


_tc1a = pl.pallas_call(
    _tc1a_body,
    grid=(NP // BR,),
    in_specs=[
        pl.BlockSpec((BR, D), lambda i: (i, 0)),
        pl.BlockSpec((D, D), lambda i: (0, 0)),
    ],
    out_specs=pl.BlockSpec((BR, D), lambda i: (i, 0)),
    out_shape=jax.ShapeDtypeStruct((NP, D), jnp.float32),
)


def _tc1b_body(graw_ref, hist_ref, g_ref, dinv_ref):
    deg = 1.0 + hist_ref[0, :, 0:1] + hist_ref[1, :, 0:1]
    dinv = lax.rsqrt(jnp.maximum(deg, 1.0))
    g_ref[...] = graw_ref[...] * dinv
    dinv_ref[...] = dinv


_tc1b = pl.pallas_call(
    _tc1b_body,
    grid=(NP // BR,),
    in_specs=[
        pl.BlockSpec((BR, D), lambda i: (i, 0)),
        pl.BlockSpec((NC, BR, D), lambda i: (0, i, 0)),
    ],
    out_specs=[
        pl.BlockSpec((BR, D), lambda i: (i, 0)),
        pl.BlockSpec((BR, 1), lambda i: (i, 0)),
    ],
    out_shape=[
        jax.ShapeDtypeStruct((NP, D), jnp.float32),
        jax.ShapeDtypeStruct((NP, 1), jnp.float32),
    ],
)


def _tc2_body(s_ref, g1_ref, dinv_ref, b_ref, w_ref, g2_ref):
    dinv = dinv_ref[...]
    h1 = jnp.maximum(dinv * (s_ref[0] + s_ref[1] + g1_ref[...]) + b_ref[...], 0.0)
    g2_ref[...] = jnp.dot(h1, w_ref[...],
                          preferred_element_type=jnp.float32, precision=_HIGH) * dinv


_tc2 = pl.pallas_call(
    _tc2_body,
    grid=(NP // BR,),
    in_specs=[
        pl.BlockSpec((NC, BR, D), lambda i: (0, i, 0)),
        pl.BlockSpec((BR, D), lambda i: (i, 0)),
        pl.BlockSpec((BR, 1), lambda i: (i, 0)),
        pl.BlockSpec((1, D), lambda i: (0, 0)),
        pl.BlockSpec((D, D), lambda i: (0, 0)),
    ],
    out_specs=pl.BlockSpec((BR, D), lambda i: (i, 0)),
    out_shape=jax.ShapeDtypeStruct((NP, D), jnp.float32),
)


def _tc3_body(s_ref, g2_ref, dinv_ref, b_ref, batch_ref, wl_ref, bl_ref, out_ref):
    dinv = dinv_ref[...]
    h2 = jnp.maximum(dinv * (s_ref[0] + s_ref[1] + g2_ref[...]) + b_ref[...], 0.0)
    oh = (lax.broadcasted_iota(jnp.int32, (G, NP), 0)
          == batch_ref[...]).astype(jnp.float32)
    sums = lax.dot_general(oh, h2, (((1,), (0,)), ((), ())),
                           preferred_element_type=jnp.float32, precision=_HIGH)
    counts = jnp.sum(oh, axis=1).reshape(G, 1)
    pooled = sums / jnp.maximum(counts, 1.0)
    out_ref[...] = jnp.dot(pooled, wl_ref[...],
                           preferred_element_type=jnp.float32, precision=_HIGH) + bl_ref[...]


_tc3 = pl.pallas_call(
    _tc3_body,
    in_specs=[
        pl.BlockSpec((NC, NP, D), lambda: (0, 0, 0)),
        pl.BlockSpec((NP, D), lambda: (0, 0)),
        pl.BlockSpec((NP, 1), lambda: (0, 0)),
        pl.BlockSpec((1, D), lambda: (0, 0)),
        pl.BlockSpec((1, NP), lambda: (0, 0)),
        pl.BlockSpec((D, 1), lambda: (0, 0)),
        pl.BlockSpec((1, 1), lambda: (0, 0)),
    ],
    out_specs=pl.BlockSpec((G, 1), lambda: (0, 0)),
    out_shape=jax.ShapeDtypeStruct((G, 1), jnp.float32),
)


def kernel(x, edge_index, batch, W1, b1, W2, b2, Wl, bl):
    x = x.astype(jnp.float32)
    src = edge_index[0]
    dst = edge_index[1]
    pad_i = jnp.arange(EPAD, dtype=src.dtype)
    src2d = jnp.concatenate([src, pad_i % 128]).reshape(EB, 128)
    dst2d = jnp.concatenate([dst, N + pad_i % (NP - N)]).reshape(EB, 128)
    x_pad = jnp.pad(x, ((0, NP - N), (0, 0)))
    batch_pad = jnp.pad(batch.astype(jnp.int32), (0, NP - N),
                        constant_values=G).reshape(1, NP)
    zeros128 = jnp.zeros((NP, D), jnp.float32)
    ones128 = jnp.ones((128, D), jnp.float32)

    hist = _deg_kernel(dst2d, ones128, zeros128)
    g1raw = _tc1a(x_pad, W1)
    g1, dinv = _tc1b(g1raw, hist)
    s1 = _spmm_kernel(g1, src2d, dst2d, zeros128)
    g2 = _tc2(s1, g1, dinv, b1.reshape(1, D), W2)
    s2 = _spmm_kernel(g2, src2d, dst2d, zeros128)
    out = _tc3(s2, g2, dinv, b2.reshape(1, D), batch_pad,
               Wl, bl.reshape(1, 1))
    return out.reshape(-1)


# speedup vs baseline: 29.0906x; 1.1922x over previous
import dataclasses
import functools

import jax
import numpy as np
import jax.numpy as jnp
from jax import lax
from jax.experimental import pallas as pl
from jax.experimental.pallas import tpu as pltpu
from jax.experimental.pallas import tpu_sc as plsc

N = 10000
E = 320000
D = 128
G = 64
NP = 10112
EB = 2560
EPAD = EB * 128 - E
NC = 2
NS = 16
WPW = EB // (NC * NS)
RPS = NP // NS

_mesh = plsc.VectorSubcoreMesh(core_axis_name="c", subcore_axis_name="s")
_HIGH = jax.lax.Precision.HIGHEST

_PAD_I = np.arange(EPAD, dtype=np.int32)
_SRC_PAD = (_PAD_I % 128).reshape(EPAD // 128, 128)
_DST_PAD = (N + _PAD_I % (NP - N)).reshape(EPAD // 128, 128).astype(np.int32)


_deg_params = pltpu.CompilerParams()
if "needs_layout_passes" in pltpu.CompilerParams.__dataclass_fields__:
    _deg_params = dataclasses.replace(_deg_params, needs_layout_passes=False)


@functools.partial(
    pl.kernel,
    out_type=jax.ShapeDtypeStruct((NC, NS, NP), jnp.float32),
    mesh=_mesh,
    scratch_types=[
        pltpu.VMEM((NP,), jnp.float32),
        pltpu.VMEM((WPW, 128), jnp.int32),
    ],
    compiler_params=_deg_params,
)
def _deg_kernel(dst_hbm, out_hbm, hist, didx):
    c = lax.axis_index("c")
    s = lax.axis_index("s")
    wid = s * NC + c

    @pl.loop(0, NP, step=16)
    def _(r):
        hist[pl.ds(r, 16)] = jnp.zeros((16,), jnp.float32)

    pltpu.sync_copy(dst_hbm.at[pl.ds(wid * WPW, WPW)], didx)
    ones = jnp.ones((16,), jnp.float32)

    @pl.loop(0, WPW)
    def _(j):
        @pl.loop(0, 128, step=16)
        def _(k):
            plsc.addupdate_scatter(hist, [didx[j, pl.ds(k, 16)]], ones)

    pltpu.sync_copy(hist, out_hbm.at[c, s])


@functools.partial(
    pl.kernel,
    out_type=jax.ShapeDtypeStruct((NC, NP, D), jnp.float32),
    mesh=_mesh,
    scratch_types=[
        pltpu.VMEM_SHARED((NP, D), jnp.float32),
        pltpu.VMEM((40, 128), jnp.int32),
        pltpu.VMEM((40, 128), jnp.int32),
        pltpu.VMEM((128, D), jnp.float32),
        pltpu.VMEM((128, D), jnp.float32),
        pltpu.SemaphoreType.DMA,
        pltpu.SemaphoreType.DMA,
    ],
)
def _spmm_kernel(g_hbm, src_hbm, dst_hbm, out_hbm, acc, sidx, didx,
                 rows0, rows1, sem0, sem1):
    c = lax.axis_index("c")
    s = lax.axis_index("s")
    wid = s * NC + c

    @pl.loop(0, 128)
    def _(r):
        @pl.loop(0, D, step=16)
        def _(k):
            rows0[r, pl.ds(k, 16)] = jnp.zeros((16,), jnp.float32)

    @pl.loop(0, 4)
    def _(i):
        pltpu.sync_copy(rows0, acc.at[pl.ds(s * RPS + i * 128, 128)])

    pltpu.sync_copy(rows0.at[pl.ds(0, 120)], acc.at[pl.ds(s * RPS + 512, 120)])
    plsc.subcore_barrier()

    @pl.loop(0, WPW // 40)
    def _(t):
        base = wid * WPW + t * 40
        pltpu.sync_copy(src_hbm.at[pl.ds(base, 40)], sidx)
        pltpu.sync_copy(dst_hbm.at[pl.ds(base, 40)], didx)
        pltpu.async_copy(g_hbm.at[sidx.at[0]], rows0, sem0)

        @pl.loop(0, 40, step=2)
        def _(j):
            pltpu.make_async_copy(g_hbm.at[sidx.at[j]], rows0, sem0).wait()
            pltpu.async_copy(g_hbm.at[sidx.at[j + 1]], rows1, sem1)
            pltpu.sync_copy(rows0, acc.at[didx.at[j]], add=True)
            pltpu.make_async_copy(g_hbm.at[sidx.at[j + 1]], rows1, sem1).wait()

            @pl.when(j + 2 < 40)
            def _():
                pltpu.async_copy(g_hbm.at[sidx.at[j + 2]], rows0, sem0)

            pltpu.sync_copy(rows1, acc.at[didx.at[j + 1]], add=True)

    plsc.subcore_barrier()
    pltpu.sync_copy(acc.at[pl.ds(s * RPS, RPS)], out_hbm.at[c, pl.ds(s * RPS, RPS)])


BR = 1264


def _tc1a_body(x_ref, w_ref, g_ref):
    g_ref[...] = jnp.dot(x_ref[...], w_ref[...],
                         preferred_element_type=jnp.float32, precision=_HIGH)


_tc1a = pl.pallas_call(
    _tc1a_body,
    grid=(NP // BR,),
    in_specs=[
        pl.BlockSpec((BR, D), lambda i: (i, 0)),
        pl.BlockSpec((D, D), lambda i: (0, 0)),
    ],
    out_specs=pl.BlockSpec((BR, D), lambda i: (i, 0)),
    out_shape=jax.ShapeDtypeStruct((NP, D), jnp.float32),
)


def _tc1b_body(graw_ref, hist_ref, g_ref, dinv_ref):
    deg = 1.0 + jnp.sum(hist_ref[...], axis=1, keepdims=True)
    dinv = lax.rsqrt(jnp.maximum(deg, 1.0))
    g_ref[...] = graw_ref[...] * dinv
    dinv_ref[...] = dinv


_tc1b = pl.pallas_call(
    _tc1b_body,
    grid=(NP // BR,),
    in_specs=[
        pl.BlockSpec((BR, D), lambda i: (i, 0)),
        pl.BlockSpec((BR, NC * NS), lambda i: (i, 0)),
    ],
    out_specs=[
        pl.BlockSpec((BR, D), lambda i: (i, 0)),
        pl.BlockSpec((BR, 1), lambda i: (i, 0)),
    ],
    out_shape=[
        jax.ShapeDtypeStruct((NP, D), jnp.float32),
        jax.ShapeDtypeStruct((NP, 1), jnp.float32),
    ],
)


def _tc2_body(s_ref, g1_ref, dinv_ref, b_ref, w_ref, g2_ref):
    dinv = dinv_ref[...]
    h1 = jnp.maximum(dinv * (s_ref[0] + s_ref[1] + g1_ref[...]) + b_ref[...], 0.0)
    g2_ref[...] = jnp.dot(h1, w_ref[...],
                          preferred_element_type=jnp.float32, precision=_HIGH) * dinv


_tc2 = pl.pallas_call(
    _tc2_body,
    grid=(NP // BR,),
    in_specs=[
        pl.BlockSpec((NC, BR, D), lambda i: (0, i, 0)),
        pl.BlockSpec((BR, D), lambda i: (i, 0)),
        pl.BlockSpec((BR, 1), lambda i: (i, 0)),
        pl.BlockSpec((1, D), lambda i: (0, 0)),
        pl.BlockSpec((D, D), lambda i: (0, 0)),
    ],
    out_specs=pl.BlockSpec((BR, D), lambda i: (i, 0)),
    out_shape=jax.ShapeDtypeStruct((NP, D), jnp.float32),
)


def _tc3_body(s_ref, g2_ref, dinv_ref, b_ref, batch_ref, wl_ref, bl_ref, out_ref):
    dinv = dinv_ref[...]
    h2 = jnp.maximum(dinv * (s_ref[0] + s_ref[1] + g2_ref[...]) + b_ref[...], 0.0)
    oh = (lax.broadcasted_iota(jnp.int32, (G, NP), 0)
          == batch_ref[...]).astype(jnp.float32)
    sums = lax.dot_general(oh, h2, (((1,), (0,)), ((), ())),
                           preferred_element_type=jnp.float32, precision=_HIGH)
    counts = jnp.sum(oh, axis=1).reshape(G, 1)
    pooled = sums / jnp.maximum(counts, 1.0)
    out_ref[...] = jnp.dot(pooled, wl_ref[...],
                           preferred_element_type=jnp.float32, precision=_HIGH) + bl_ref[...]


_tc3 = pl.pallas_call(
    _tc3_body,
    in_specs=[
        pl.BlockSpec((NC, NP, D), lambda: (0, 0, 0)),
        pl.BlockSpec((NP, D), lambda: (0, 0)),
        pl.BlockSpec((NP, 1), lambda: (0, 0)),
        pl.BlockSpec((1, D), lambda: (0, 0)),
        pl.BlockSpec((1, NP), lambda: (0, 0)),
        pl.BlockSpec((D, 1), lambda: (0, 0)),
        pl.BlockSpec((1, 1), lambda: (0, 0)),
    ],
    out_specs=pl.BlockSpec((G, 1), lambda: (0, 0)),
    out_shape=jax.ShapeDtypeStruct((G, 1), jnp.float32),
)


def kernel(x, edge_index, batch, W1, b1, W2, b2, Wl, bl):
    x = x.astype(jnp.float32)
    src = edge_index[0]
    dst = edge_index[1]
    src2d = jnp.concatenate([src.reshape(E // 128, 128), _SRC_PAD])
    dst2d = jnp.concatenate([dst.reshape(E // 128, 128), _DST_PAD])
    x_pad = jnp.pad(x, ((0, NP - N), (0, 0)))
    batch_pad = jnp.pad(batch.astype(jnp.int32), (0, NP - N),
                        constant_values=G).reshape(1, NP)

    hist = _deg_kernel(dst2d)
    g1raw = _tc1a(x_pad, W1)
    g1, dinv = _tc1b(g1raw, hist.reshape(NC * NS, NP).T)
    s1 = _spmm_kernel(g1, src2d, dst2d)
    g2 = _tc2(s1, g1, dinv, b1.reshape(1, D), W2)
    s2 = _spmm_kernel(g2, src2d, dst2d)
    out = _tc3(s2, g2, dinv, b2.reshape(1, D), batch_pad,
               Wl, bl.reshape(1, 1))
    return out.reshape(-1)


# speedup vs baseline: 29.5701x; 1.0165x over previous
import dataclasses
import functools

import jax
import numpy as np
import jax.numpy as jnp
from jax import lax
from jax.experimental import pallas as pl
from jax.experimental.pallas import tpu as pltpu
from jax.experimental.pallas import tpu_sc as plsc

N = 10000
E = 320000
D = 128
G = 64
NP = 10112
EB = 2560
EPAD = EB * 128 - E
NC = 2
NS = 16
WPW = EB // (NC * NS)
RPS = NP // NS

_mesh = plsc.VectorSubcoreMesh(core_axis_name="c", subcore_axis_name="s")
_HIGH = jax.lax.Precision.HIGHEST

_PAD_I = np.arange(EPAD, dtype=np.int32)
_SRC_PAD = (_PAD_I % 128).reshape(EPAD // 128, 128)
_DST_PAD = (N + _PAD_I % (NP - N)).reshape(EPAD // 128, 128).astype(np.int32)


_deg_params = pltpu.CompilerParams()
if "needs_layout_passes" in pltpu.CompilerParams.__dataclass_fields__:
    _deg_params = dataclasses.replace(_deg_params, needs_layout_passes=False)


@functools.partial(
    pl.kernel,
    out_type=jax.ShapeDtypeStruct((NC, NS, NP), jnp.float32),
    mesh=_mesh,
    scratch_types=[
        pltpu.VMEM((NP,), jnp.float32),
        pltpu.VMEM((WPW, 128), jnp.int32),
    ],
    compiler_params=_deg_params,
)
def _deg_kernel(dst_hbm, out_hbm, hist, didx):
    c = lax.axis_index("c")
    s = lax.axis_index("s")
    wid = s * NC + c

    @pl.loop(0, NP, step=16)
    def _(r):
        hist[pl.ds(r, 16)] = jnp.zeros((16,), jnp.float32)

    pltpu.sync_copy(dst_hbm.at[pl.ds(wid * WPW, WPW)], didx)
    ones = jnp.ones((16,), jnp.float32)

    @pl.loop(0, WPW)
    def _(j):
        @pl.loop(0, 128, step=16)
        def _(k):
            plsc.addupdate_scatter(hist, [didx[j, pl.ds(k, 16)]], ones)

    pltpu.sync_copy(hist, out_hbm.at[c, s])


@functools.partial(
    pl.kernel,
    out_type=jax.ShapeDtypeStruct((NC, NP, D), jnp.float32),
    mesh=_mesh,
    scratch_types=[
        pltpu.VMEM_SHARED((NP, D), jnp.float32),
        pltpu.VMEM((40, 128), jnp.int32),
        pltpu.VMEM((40, 128), jnp.int32),
        pltpu.VMEM((128, D), jnp.float32),
        pltpu.VMEM((128, D), jnp.float32),
        pltpu.SemaphoreType.DMA,
        pltpu.SemaphoreType.DMA,
    ],
)
def _spmm_kernel(g_hbm, src_hbm, dst_hbm, out_hbm, acc, sidx, didx,
                 rows0, rows1, sem0, sem1):
    c = lax.axis_index("c")
    s = lax.axis_index("s")
    wid = s * NC + c

    @pl.loop(0, 128)
    def _(r):
        @pl.loop(0, D, step=16)
        def _(k):
            rows0[r, pl.ds(k, 16)] = jnp.zeros((16,), jnp.float32)

    @pl.loop(0, 4)
    def _(i):
        pltpu.sync_copy(rows0, acc.at[pl.ds(s * RPS + i * 128, 128)])

    pltpu.sync_copy(rows0.at[pl.ds(0, 120)], acc.at[pl.ds(s * RPS + 512, 120)])
    plsc.subcore_barrier()

    @pl.loop(0, WPW // 40)
    def _(t):
        base = wid * WPW + t * 40
        pltpu.sync_copy(src_hbm.at[pl.ds(base, 40)], sidx)
        pltpu.sync_copy(dst_hbm.at[pl.ds(base, 40)], didx)
        pltpu.async_copy(g_hbm.at[sidx.at[0]], rows0, sem0)

        @pl.loop(0, 40, step=2)
        def _(j):
            pltpu.make_async_copy(g_hbm.at[sidx.at[j]], rows0, sem0).wait()
            pltpu.async_copy(g_hbm.at[sidx.at[j + 1]], rows1, sem1)
            pltpu.sync_copy(rows0, acc.at[didx.at[j]], add=True)
            pltpu.make_async_copy(g_hbm.at[sidx.at[j + 1]], rows1, sem1).wait()

            @pl.when(j + 2 < 40)
            def _():
                pltpu.async_copy(g_hbm.at[sidx.at[j + 2]], rows0, sem0)

            pltpu.sync_copy(rows1, acc.at[didx.at[j + 1]], add=True)

    plsc.subcore_barrier()
    pltpu.sync_copy(acc.at[pl.ds(s * RPS, RPS)], out_hbm.at[c, pl.ds(s * RPS, RPS)])


BR = 2528


def _tc1a_body(x_ref, w_ref, g_ref):
    g_ref[...] = jnp.dot(x_ref[...], w_ref[...],
                         preferred_element_type=jnp.float32, precision=_HIGH)


_tc1a = pl.pallas_call(
    _tc1a_body,
    grid=(NP // BR,),
    in_specs=[
        pl.BlockSpec((BR, D), lambda i: (i, 0)),
        pl.BlockSpec((D, D), lambda i: (0, 0)),
    ],
    out_specs=pl.BlockSpec((BR, D), lambda i: (i, 0)),
    out_shape=jax.ShapeDtypeStruct((NP, D), jnp.float32),
)


def _tc1b_body(graw_ref, hist_ref, g_ref, dinv_ref):
    deg = 1.0 + jnp.sum(hist_ref[...], axis=1, keepdims=True)
    dinv = lax.rsqrt(jnp.maximum(deg, 1.0))
    g_ref[...] = graw_ref[...] * dinv
    dinv_ref[...] = dinv


_tc1b = pl.pallas_call(
    _tc1b_body,
    grid=(NP // BR,),
    in_specs=[
        pl.BlockSpec((BR, D), lambda i: (i, 0)),
        pl.BlockSpec((BR, NC * NS), lambda i: (i, 0)),
    ],
    out_specs=[
        pl.BlockSpec((BR, D), lambda i: (i, 0)),
        pl.BlockSpec((BR, 1), lambda i: (i, 0)),
    ],
    out_shape=[
        jax.ShapeDtypeStruct((NP, D), jnp.float32),
        jax.ShapeDtypeStruct((NP, 1), jnp.float32),
    ],
)


def _tc2_body(s_ref, g1_ref, dinv_ref, b_ref, w_ref, g2_ref):
    dinv = dinv_ref[...]
    h1 = jnp.maximum(dinv * (s_ref[0] + s_ref[1] + g1_ref[...]) + b_ref[...], 0.0)
    g2_ref[...] = jnp.dot(h1, w_ref[...],
                          preferred_element_type=jnp.float32, precision=_HIGH) * dinv


_tc2 = pl.pallas_call(
    _tc2_body,
    grid=(NP // BR,),
    in_specs=[
        pl.BlockSpec((NC, BR, D), lambda i: (0, i, 0)),
        pl.BlockSpec((BR, D), lambda i: (i, 0)),
        pl.BlockSpec((BR, 1), lambda i: (i, 0)),
        pl.BlockSpec((1, D), lambda i: (0, 0)),
        pl.BlockSpec((D, D), lambda i: (0, 0)),
    ],
    out_specs=pl.BlockSpec((BR, D), lambda i: (i, 0)),
    out_shape=jax.ShapeDtypeStruct((NP, D), jnp.float32),
)


def _tc3_body(s_ref, g2_ref, dinv_ref, b_ref, batch_ref, wl_ref, bl_ref, out_ref):
    dinv = dinv_ref[...]
    h2 = jnp.maximum(dinv * (s_ref[0] + s_ref[1] + g2_ref[...]) + b_ref[...], 0.0)
    h2 = jnp.where(lax.broadcasted_iota(jnp.int32, (NP, 1), 0) < N, h2, 0.0)
    oh = (lax.broadcasted_iota(jnp.int32, (G, NP), 0)
          == batch_ref[...]).astype(jnp.float32)
    sums = lax.dot_general(oh, h2, (((1,), (0,)), ((), ())),
                           preferred_element_type=jnp.float32, precision=_HIGH)
    counts = jnp.sum(oh, axis=1).reshape(G, 1)
    pooled = sums / jnp.maximum(counts, 1.0)
    out_ref[...] = jnp.dot(pooled, wl_ref[...],
                           preferred_element_type=jnp.float32, precision=_HIGH) + bl_ref[...]


_tc3 = pl.pallas_call(
    _tc3_body,
    in_specs=[
        pl.BlockSpec((NC, NP, D), lambda: (0, 0, 0)),
        pl.BlockSpec((NP, D), lambda: (0, 0)),
        pl.BlockSpec((NP, 1), lambda: (0, 0)),
        pl.BlockSpec((1, D), lambda: (0, 0)),
        pl.BlockSpec((1, NP), lambda: (0, 0)),
        pl.BlockSpec((D, 1), lambda: (0, 0)),
        pl.BlockSpec((1, 1), lambda: (0, 0)),
    ],
    out_specs=pl.BlockSpec((G, 1), lambda: (0, 0)),
    out_shape=jax.ShapeDtypeStruct((G, 1), jnp.float32),
)


def kernel(x, edge_index, batch, W1, b1, W2, b2, Wl, bl):
    x = x.astype(jnp.float32)
    src = edge_index[0]
    dst = edge_index[1]
    src2d = jnp.concatenate([src.reshape(E // 128, 128), _SRC_PAD])
    dst2d = jnp.concatenate([dst.reshape(E // 128, 128), _DST_PAD])
    x_pad = jnp.pad(x, ((0, NP - N), (0, 0)))
    batch_pad = jnp.pad(batch.astype(jnp.int32), (0, NP - N),
                        constant_values=G).reshape(1, NP)

    hist = _deg_kernel(dst2d)
    g1raw = _tc1a(x_pad, W1)
    g1, dinv = _tc1b(g1raw, hist.reshape(NC * NS, NP).T)
    s1 = _spmm_kernel(g1, src2d, dst2d)
    g2 = _tc2(s1, g1, dinv, b1.reshape(1, D), W2)
    s2 = _spmm_kernel(g2, src2d, dst2d)
    out = _tc3(s2, g2, dinv, b2.reshape(1, D), batch_pad,
               Wl, bl.reshape(1, 1))
    return out.reshape(-1)
